# Initial kernel scaffold; baseline (speedup 1.0000x reference)
import jax
import jax.numpy as jnp
from jax.experimental import pallas as pl


def kernel(points, params):
    raise NotImplementedError("write your pallas kernel here")


# speedup vs baseline: 10.6144x; 10.6144x over previous
import functools

import jax
import jax.numpy as jnp
from jax import lax
from jax.experimental import pallas as pl
from jax.experimental.pallas import tpu as pltpu
from jax.experimental.pallas import tpu_sc as plsc

KNN = 10
N = 1024
B = 32
P = B * N
NK = N * KNN
DP = 128
EPS = 1e-5
NW = 32
CCH = 32
CK = CCH * KNN


def _lrelu(x):
    return jnp.where(x >= 0, x, 0.2 * x)


def _dot(a, b, dims, prec=None):
    return lax.dot_general(a, b, dimension_numbers=(dims, ((), ())),
                           preferred_element_type=jnp.float32,
                           precision=prec)


def _bn_act(x, m_ref, sv_ref, g_ref, b_ref):
    return _lrelu((x - m_ref[...]) / sv_ref[...] * g_ref[...] + b_ref[...])


def _prep_core(xp, b):
    n = xp.shape[0]
    g_def = _dot(xp, xp, ((1,), (1,)))
    g_hi = _dot(xp, xp, ((1,), (1,)), lax.Precision.HIGHEST)
    rowi = lax.broadcasted_iota(jnp.int32, (n, n), 0)
    coli = lax.broadcasted_iota(jnp.int32, (n, n), 1)
    eye = rowi == coli
    sqr = jnp.sum(jnp.where(eye, g_hi, 0.0), axis=0, keepdims=True)
    d = sqr - 2.0 * g_def
    idx16 = jnp.zeros((n, 16), jnp.int32)
    lane16 = lax.broadcasted_iota(jnp.int32, (n, 16), 1)
    big = jnp.float32(3.0e38)
    for it in range(KNN + 1):
        rmin = jnp.min(d, axis=1, keepdims=True)
        amin = jnp.min(jnp.where(d == rmin, coli, n), axis=1, keepdims=True)
        if it > 0:
            idx16 = jnp.where(lane16 == (it - 1), amin, idx16)
        d = jnp.where(coli == amin, big, d)
    return idx16 + b * jnp.int32(N)


def _pad_dp(x):
    n, dd = x.shape
    if dd < DP:
        x = jnp.concatenate([x, jnp.zeros((n, DP - dd), jnp.float32)], axis=1)
    return x


def _prep_body_first(x_ref, xout_ref, idx_ref):
    b = pl.program_id(0)
    xp = _pad_dp(x_ref[0])
    xout_ref[0] = xp
    idx_ref[0] = _prep_core(xp, b)


def _prep_body_act(ym_ref, m_ref, sv_ref, g_ref, b_ref, xout_ref, idx_ref):
    b = pl.program_id(0)
    xp = _pad_dp(_bn_act(ym_ref[0], m_ref, sv_ref, g_ref, b_ref))
    xout_ref[0] = xp
    idx_ref[0] = _prep_core(xp, b)


def _prep_call(inputs, d, first):
    body = _prep_body_first if first else _prep_body_act
    in_specs = [pl.BlockSpec((1, N, d), lambda b: (b, 0, 0))]
    if not first:
        in_specs += [pl.BlockSpec((1, d), lambda b: (0, 0))] * 4
    return pl.pallas_call(
        body,
        grid=(B,),
        in_specs=in_specs,
        out_specs=[pl.BlockSpec((1, N, DP), lambda b: (b, 0, 0)),
                   pl.BlockSpec((1, N, 16), lambda b: (b, 0, 0))],
        out_shape=[jax.ShapeDtypeStruct((B, N, DP), jnp.float32),
                   jax.ShapeDtypeStruct((B, N, 16), jnp.int32)],
    )(*inputs)


def _sc_gather_edges(x_flat, idx_flat, fw):
    half = fw // 2
    ppw = P // NW
    nch = ppw // CCH
    subs = []
    off = 0
    while off < CK:
        ln = min(128, CK - off)
        subs.append((off, ln))
        off += ln
    mesh = plsc.VectorSubcoreMesh(core_axis_name="c", subcore_axis_name="s")

    @functools.partial(
        pl.kernel, mesh=mesh,
        out_type=jax.ShapeDtypeStruct((P * KNN, fw), jnp.float32),
        scratch_types=[pltpu.VMEM((CK,), jnp.int32),
                       pltpu.VMEM((CK, DP), jnp.float32),
                       pltpu.VMEM((CCH, DP), jnp.float32),
                       pltpu.VMEM((CK, fw), jnp.float32),
                       pltpu.SemaphoreType.DMA],
    )
    def k(x_hbm, idx_hbm, f_hbm, idx_v, rows_v, ctr_v, f_v, sem):
        wid = lax.axis_index("s") * 2 + lax.axis_index("c")

        def chunk(gi, carry):
            p0 = wid * ppw + gi * CCH
            pltpu.sync_copy(idx_hbm.at[pl.ds(p0 * KNN, CK)], idx_v)
            cops = [pltpu.async_copy(x_hbm.at[idx_v.at[pl.ds(soff, sln)]],
                                     rows_v.at[pl.ds(soff, sln)], sem)
                    for (soff, sln) in subs]
            pltpu.sync_copy(x_hbm.at[pl.ds(p0, CCH)], ctr_v)
            for cop in cops:
                cop.wait()

            def pt(i, c2):
                for v in range(half // 16):
                    sl = pl.ds(v * 16, 16)
                    sh = pl.ds(half + v * 16, 16)
                    c16 = ctr_v[i, sl]
                    for j in range(KNN):
                        e = i * KNN + j
                        f_v[e, sl] = rows_v[e, sl] - c16
                        f_v[e, sh] = c16
                return c2

            lax.fori_loop(0, CCH, pt, 0)
            pltpu.sync_copy(f_v, f_hbm.at[pl.ds(p0 * KNN, CK)])
            return carry

        lax.fori_loop(0, nch, chunk, 0)

    return k(x_flat, idx_flat)


def _edge_body(f_ref, wf_ref, ym_ref, st_ref):
    y = _dot(f_ref[0], wf_ref[...], ((1,), (0,)))
    o = y.shape[1]
    ym_ref[0] = jnp.max(y.reshape(N, KNN, o), axis=1)
    s1 = jnp.sum(y, axis=0, keepdims=True)
    c = s1 / float(NK)
    s2c = jnp.sum((y - c) ** 2, axis=0, keepdims=True)
    st_ref[0] = jnp.concatenate([s1, s2c, c,
                                 jnp.zeros((5, o), jnp.float32)], axis=0)


def _edge_call(f3, wf, o):
    fw = wf.shape[0]
    return pl.pallas_call(
        _edge_body,
        grid=(B,),
        in_specs=[pl.BlockSpec((1, NK, fw), lambda b: (b, 0, 0)),
                  pl.BlockSpec((fw, o), lambda b: (0, 0))],
        out_specs=[pl.BlockSpec((1, N, o), lambda b: (b, 0, 0)),
                   pl.BlockSpec((1, 8, o), lambda b: (b, 0, 0))],
        out_shape=[jax.ShapeDtypeStruct((B, N, o), jnp.float32),
                   jax.ShapeDtypeStruct((B, 8, o), jnp.float32)],
    )(f3, wf)


def _conv5_body(y1, m1, v1, g1, b1, y2, m2, v2, g2, b2,
                y3, m3, v3, g3, b3, y4, m4, v4, g4, b4, w5t_ref, out_ref):
    xs = [_bn_act(y1[0], m1, v1, g1, b1), _bn_act(y2[0], m2, v2, g2, b2),
          _bn_act(y3[0], m3, v3, g3, b3), _bn_act(y4[0], m4, v4, g4, b4)]
    xcat = jnp.concatenate(xs, axis=1)
    y = _dot(xcat, w5t_ref[...], ((1,), (0,)))
    ymax = jnp.max(y, axis=0, keepdims=True)
    s1 = jnp.sum(y, axis=0, keepdims=True)
    c = s1 / float(N)
    s2c = jnp.sum((y - c) ** 2, axis=0, keepdims=True)
    out_ref[0] = jnp.concatenate(
        [ymax, s1, s2c, c, jnp.zeros((4, y.shape[1]), jnp.float32)], axis=0)


def _conv5_call(yms, ms, svs, gs, bs, w5t):
    dims = [64, 64, 128, 256]
    in_specs = []
    inputs = []
    for l in range(4):
        d = dims[l]
        in_specs += [pl.BlockSpec((1, N, d), lambda b: (b, 0, 0))]
        in_specs += [pl.BlockSpec((1, d), lambda b: (0, 0))] * 4
        inputs += [yms[l], ms[l], svs[l], gs[l], bs[l]]
    in_specs += [pl.BlockSpec((512, 1024), lambda b: (0, 0))]
    inputs += [w5t]
    return pl.pallas_call(
        _conv5_body,
        grid=(B,),
        in_specs=in_specs,
        out_specs=pl.BlockSpec((1, 8, 1024), lambda b: (b, 0, 0)),
        out_shape=jax.ShapeDtypeStruct((B, 8, 1024), jnp.float32),
    )(*inputs)


def _fc_body(ym_ref, m5_ref, v5_ref, g5_ref, b5_ref, w1_ref, g1_ref, b1_ref,
             w2_ref, c2b_ref, g2_ref, b2_ref, w3_ref, c3b_ref, out_ref):
    y = _bn_act(ym_ref[...], m5_ref, v5_ref, g5_ref, b5_ref)
    h = _dot(y, w1_ref[...], ((1,), (0,)))
    m = jnp.mean(h, axis=0, keepdims=True)
    v = jnp.mean((h - m) ** 2, axis=0, keepdims=True)
    h = _lrelu((h - m) / jnp.sqrt(v + EPS) * g1_ref[...] + b1_ref[...])
    h = _dot(h, w2_ref[...], ((1,), (0,))) + c2b_ref[...]
    m = jnp.mean(h, axis=0, keepdims=True)
    v = jnp.mean((h - m) ** 2, axis=0, keepdims=True)
    h = _lrelu((h - m) / jnp.sqrt(v + EPS) * g2_ref[...] + b2_ref[...])
    out_ref[...] = _dot(h, w3_ref[...], ((1,), (0,))) + c3b_ref[...]


def _fc_call(ymax, m5, sv5, params):
    w3t = jnp.zeros((256, 128), jnp.float32).at[:, :40].set(params['fW3'].T)
    c3b = jnp.zeros((1, 128), jnp.float32).at[:, :40].set(
        params['fc3b'].reshape(1, -1))
    out = pl.pallas_call(
        _fc_body,
        out_shape=jax.ShapeDtypeStruct((B, 128), jnp.float32),
    )(ymax, m5, sv5, params['g5'].reshape(1, -1), params['b5'].reshape(1, -1),
      params['fW1'].T, params['fg1'].reshape(1, -1), params['fb1'].reshape(1, -1),
      params['fW2'].T, params['fc2b'].reshape(1, -1),
      params['fg2'].reshape(1, -1), params['fb2'].reshape(1, -1),
      w3t, c3b)
    return out[:, :40]


def _combine_stats(s1_b, s2c_b, c_b, nb):
    t = nb * B
    m = jnp.sum(s1_b, axis=0) / t
    r_b = s1_b - nb * c_b
    dc = c_b - m[None, :]
    v = jnp.sum(s2c_b + 2.0 * dc * r_b + nb * dc * dc, axis=0) / t
    return m, v


def _pack_w(w, d, fw):
    half = fw // 2
    o = w.shape[0]
    wf = jnp.zeros((fw, o), jnp.float32)
    wf = wf.at[:d, :].set(w[:, :d].T)
    wf = wf.at[half:half + d, :].set(w[:, d:].T)
    return wf


_LAYERS = (('W1', 'g1', 'b1', 3, 64, 32),
           ('W2', 'g2', 'b2', 64, 64, 128),
           ('W3', 'g3', 'b3', 64, 128, 128),
           ('W4', 'g4', 'b4', 128, 256, 256))


def kernel(points, params):
    yms, ms, svs, gs, bs = [], [], [], [], []
    xout = idx16 = None
    for li, (wn, gn, bnm, d, o, fw) in enumerate(_LAYERS):
        if li == 0:
            xout, idx16 = _prep_call((points,), d, True)
        else:
            xout, idx16 = _prep_call(
                (yms[-1], ms[-1], svs[-1], gs[-1], bs[-1]), d, False)
        idx_flat = idx16[:, :, :KNN].reshape(-1)
        f_flat = _sc_gather_edges(xout.reshape(P, DP), idx_flat, fw)
        wf = _pack_w(params[wn], d, fw)
        ymax, st = _edge_call(f_flat.reshape(B, NK, fw), wf, o)
        m, v = _combine_stats(st[:, 0], st[:, 1], st[:, 2], float(NK))
        yms.append(ymax)
        ms.append(m.reshape(1, -1))
        svs.append(jnp.sqrt(v + EPS).reshape(1, -1))
        gs.append(params[gn].reshape(1, -1))
        bs.append(params[bnm].reshape(1, -1))

    stats5 = _conv5_call(yms, ms, svs, gs, bs, params['W5'].T)
    ymax5 = stats5[:, 0, :]
    m5, v5 = _combine_stats(stats5[:, 1], stats5[:, 2], stats5[:, 3], float(N))
    return _fc_call(ymax5, m5.reshape(1, -1), jnp.sqrt(v5 + EPS).reshape(1, -1),
                    params)


# speedup vs baseline: 11.9547x; 1.1263x over previous
import functools

import jax
import jax.numpy as jnp
from jax import lax
from jax.experimental import pallas as pl
from jax.experimental.pallas import tpu as pltpu
from jax.experimental.pallas import tpu_sc as plsc

KNN = 10
N = 1024
B = 32
P = B * N
NK = N * KNN
DP = 128
EPS = 1e-5
NW = 32
CCH = 32
CK = CCH * KNN


def _lrelu(x):
    return jnp.where(x >= 0, x, 0.2 * x)


def _dot(a, b, dims, prec=None):
    return lax.dot_general(a, b, dimension_numbers=(dims, ((), ())),
                           preferred_element_type=jnp.float32,
                           precision=prec)


def _bn_act(x, m_ref, sv_ref, g_ref, b_ref):
    return _lrelu((x - m_ref[...]) / sv_ref[...] * g_ref[...] + b_ref[...])


def _prep_core(xp, b):
    n = xp.shape[0]
    g_def = _dot(xp, xp, ((1,), (1,)))
    coli = lax.broadcasted_iota(jnp.int32, (n, n), 1)
    sqc = jnp.sum(xp * xp, axis=1, keepdims=True)
    sqr = jnp.transpose(sqc)
    d = (sqc + sqr) - 2.0 * g_def
    idx16 = jnp.zeros((n, 16), jnp.int32)
    lane16 = lax.broadcasted_iota(jnp.int32, (n, 16), 1)
    big = jnp.float32(3.0e38)
    for it in range(KNN + 1):
        rmin = jnp.min(d, axis=1, keepdims=True)
        amin = jnp.min(jnp.where(d == rmin, coli, n), axis=1, keepdims=True)
        if it > 0:
            idx16 = jnp.where(lane16 == (it - 1), amin, idx16)
        d = jnp.where(coli == amin, big, d)
    return idx16 + b * jnp.int32(N)


def _pad_dp(x):
    n, dd = x.shape
    if dd < DP:
        x = jnp.concatenate([x, jnp.zeros((n, DP - dd), jnp.float32)], axis=1)
    return x


def _prep_body_first(x_ref, xout_ref, idx_ref):
    b = pl.program_id(0)
    xp = _pad_dp(x_ref[0])
    xout_ref[0] = xp
    idx_ref[0] = _prep_core(xp, b)


def _prep_body_act(ym_ref, m_ref, sv_ref, g_ref, b_ref, xout_ref, idx_ref):
    b = pl.program_id(0)
    xp = _pad_dp(_bn_act(ym_ref[0], m_ref, sv_ref, g_ref, b_ref))
    xout_ref[0] = xp
    idx_ref[0] = _prep_core(xp, b)


def _prep_call(inputs, d, first):
    body = _prep_body_first if first else _prep_body_act
    in_specs = [pl.BlockSpec((1, N, d), lambda b: (b, 0, 0))]
    if not first:
        in_specs += [pl.BlockSpec((1, d), lambda b: (0, 0))] * 4
    return pl.pallas_call(
        body,
        grid=(B,),
        in_specs=in_specs,
        out_specs=[pl.BlockSpec((1, N, DP), lambda b: (b, 0, 0)),
                   pl.BlockSpec((1, N, 16), lambda b: (b, 0, 0))],
        out_shape=[jax.ShapeDtypeStruct((B, N, DP), jnp.float32),
                   jax.ShapeDtypeStruct((B, N, 16), jnp.int32)],
    )(*inputs)


def _sc_gather_edges(x_flat, idx_flat, fw):
    half = fw // 2
    ppw = P // NW
    nch = ppw // CCH
    subs = []
    off = 0
    while off < CK:
        ln = min(128, CK - off)
        subs.append((off, ln))
        off += ln
    mesh = plsc.VectorSubcoreMesh(core_axis_name="c", subcore_axis_name="s")

    @functools.partial(
        pl.kernel, mesh=mesh,
        out_type=jax.ShapeDtypeStruct((P * KNN, fw), jnp.float32),
        scratch_types=[pltpu.VMEM((CK,), jnp.int32),
                       pltpu.VMEM((CK, DP), jnp.float32),
                       pltpu.VMEM((CCH, DP), jnp.float32),
                       pltpu.VMEM((CK, fw), jnp.float32),
                       pltpu.SemaphoreType.DMA],
    )
    def k(x_hbm, idx_hbm, f_hbm, idx_v, rows_v, ctr_v, f_v, sem):
        wid = lax.axis_index("s") * 2 + lax.axis_index("c")

        def chunk(gi, carry):
            p0 = wid * ppw + gi * CCH
            pltpu.sync_copy(idx_hbm.at[pl.ds(p0 * KNN, CK)], idx_v)
            cops = [pltpu.async_copy(x_hbm.at[idx_v.at[pl.ds(soff, sln)]],
                                     rows_v.at[pl.ds(soff, sln)], sem)
                    for (soff, sln) in subs]
            pltpu.sync_copy(x_hbm.at[pl.ds(p0, CCH)], ctr_v)
            for cop in cops:
                cop.wait()

            def pt(i, c2):
                for v in range(half // 16):
                    sl = pl.ds(v * 16, 16)
                    sh = pl.ds(half + v * 16, 16)
                    c16 = ctr_v[i, sl]
                    for j in range(KNN):
                        e = i * KNN + j
                        f_v[e, sl] = rows_v[e, sl] - c16
                        f_v[e, sh] = c16
                return c2

            lax.fori_loop(0, CCH, pt, 0)
            pltpu.sync_copy(f_v, f_hbm.at[pl.ds(p0 * KNN, CK)])
            return carry

        lax.fori_loop(0, nch, chunk, 0)

    return k(x_flat, idx_flat)


def _edge_body(f_ref, wf_ref, ym_ref, st_ref):
    y = _dot(f_ref[0], wf_ref[...], ((1,), (0,)))
    o = y.shape[1]
    ym_ref[0] = jnp.max(y.reshape(N, KNN, o), axis=1)
    s1 = jnp.sum(y, axis=0, keepdims=True)
    c = s1 / float(NK)
    s2c = jnp.sum((y - c) ** 2, axis=0, keepdims=True)
    st_ref[0] = jnp.concatenate([s1, s2c, c,
                                 jnp.zeros((5, o), jnp.float32)], axis=0)


def _edge_call(f3, wf, o):
    fw = wf.shape[0]
    return pl.pallas_call(
        _edge_body,
        grid=(B,),
        in_specs=[pl.BlockSpec((1, NK, fw), lambda b: (b, 0, 0)),
                  pl.BlockSpec((fw, o), lambda b: (0, 0))],
        out_specs=[pl.BlockSpec((1, N, o), lambda b: (b, 0, 0)),
                   pl.BlockSpec((1, 8, o), lambda b: (b, 0, 0))],
        out_shape=[jax.ShapeDtypeStruct((B, N, o), jnp.float32),
                   jax.ShapeDtypeStruct((B, 8, o), jnp.float32)],
    )(f3, wf)


def _conv5_body(y1, m1, v1, g1, b1, y2, m2, v2, g2, b2,
                y3, m3, v3, g3, b3, y4, m4, v4, g4, b4, w5t_ref, out_ref):
    xs = [_bn_act(y1[0], m1, v1, g1, b1), _bn_act(y2[0], m2, v2, g2, b2),
          _bn_act(y3[0], m3, v3, g3, b3), _bn_act(y4[0], m4, v4, g4, b4)]
    xcat = jnp.concatenate(xs, axis=1)
    y = _dot(xcat, w5t_ref[...], ((1,), (0,)))
    ymax = jnp.max(y, axis=0, keepdims=True)
    s1 = jnp.sum(y, axis=0, keepdims=True)
    c = s1 / float(N)
    s2c = jnp.sum((y - c) ** 2, axis=0, keepdims=True)
    out_ref[0] = jnp.concatenate(
        [ymax, s1, s2c, c, jnp.zeros((4, y.shape[1]), jnp.float32)], axis=0)


def _conv5_call(yms, ms, svs, gs, bs, w5t):
    dims = [64, 64, 128, 256]
    in_specs = []
    inputs = []
    for l in range(4):
        d = dims[l]
        in_specs += [pl.BlockSpec((1, N, d), lambda b: (b, 0, 0))]
        in_specs += [pl.BlockSpec((1, d), lambda b: (0, 0))] * 4
        inputs += [yms[l], ms[l], svs[l], gs[l], bs[l]]
    in_specs += [pl.BlockSpec((512, 1024), lambda b: (0, 0))]
    inputs += [w5t]
    return pl.pallas_call(
        _conv5_body,
        grid=(B,),
        in_specs=in_specs,
        out_specs=pl.BlockSpec((1, 8, 1024), lambda b: (b, 0, 0)),
        out_shape=jax.ShapeDtypeStruct((B, 8, 1024), jnp.float32),
    )(*inputs)


def _fc_body(ym_ref, m5_ref, v5_ref, g5_ref, b5_ref, w1_ref, g1_ref, b1_ref,
             w2_ref, c2b_ref, g2_ref, b2_ref, w3_ref, c3b_ref, out_ref):
    y = _bn_act(ym_ref[...], m5_ref, v5_ref, g5_ref, b5_ref)
    h = _dot(y, w1_ref[...], ((1,), (0,)))
    m = jnp.mean(h, axis=0, keepdims=True)
    v = jnp.mean((h - m) ** 2, axis=0, keepdims=True)
    h = _lrelu((h - m) / jnp.sqrt(v + EPS) * g1_ref[...] + b1_ref[...])
    h = _dot(h, w2_ref[...], ((1,), (0,))) + c2b_ref[...]
    m = jnp.mean(h, axis=0, keepdims=True)
    v = jnp.mean((h - m) ** 2, axis=0, keepdims=True)
    h = _lrelu((h - m) / jnp.sqrt(v + EPS) * g2_ref[...] + b2_ref[...])
    out_ref[...] = _dot(h, w3_ref[...], ((1,), (0,))) + c3b_ref[...]


def _fc_call(ymax, m5, sv5, params):
    w3t = jnp.zeros((256, 128), jnp.float32).at[:, :40].set(params['fW3'].T)
    c3b = jnp.zeros((1, 128), jnp.float32).at[:, :40].set(
        params['fc3b'].reshape(1, -1))
    out = pl.pallas_call(
        _fc_body,
        out_shape=jax.ShapeDtypeStruct((B, 128), jnp.float32),
    )(ymax, m5, sv5, params['g5'].reshape(1, -1), params['b5'].reshape(1, -1),
      params['fW1'].T, params['fg1'].reshape(1, -1), params['fb1'].reshape(1, -1),
      params['fW2'].T, params['fc2b'].reshape(1, -1),
      params['fg2'].reshape(1, -1), params['fb2'].reshape(1, -1),
      w3t, c3b)
    return out[:, :40]


def _combine_stats(s1_b, s2c_b, c_b, nb):
    t = nb * B
    m = jnp.sum(s1_b, axis=0) / t
    r_b = s1_b - nb * c_b
    dc = c_b - m[None, :]
    v = jnp.sum(s2c_b + 2.0 * dc * r_b + nb * dc * dc, axis=0) / t
    return m, v


def _pack_w(w, d, fw):
    half = fw // 2
    o = w.shape[0]
    wf = jnp.zeros((fw, o), jnp.float32)
    wf = wf.at[:d, :].set(w[:, :d].T)
    wf = wf.at[half:half + d, :].set(w[:, d:].T)
    return wf


_LAYERS = (('W1', 'g1', 'b1', 3, 64, 32),
           ('W2', 'g2', 'b2', 64, 64, 128),
           ('W3', 'g3', 'b3', 64, 128, 128),
           ('W4', 'g4', 'b4', 128, 256, 256))


def kernel(points, params):
    yms, ms, svs, gs, bs = [], [], [], [], []
    xout = idx16 = None
    for li, (wn, gn, bnm, d, o, fw) in enumerate(_LAYERS):
        if li == 0:
            xout, idx16 = _prep_call((points,), d, True)
        else:
            xout, idx16 = _prep_call(
                (yms[-1], ms[-1], svs[-1], gs[-1], bs[-1]), d, False)
        idx_flat = idx16[:, :, :KNN].reshape(-1)
        f_flat = _sc_gather_edges(xout.reshape(P, DP), idx_flat, fw)
        wf = _pack_w(params[wn], d, fw)
        ymax, st = _edge_call(f_flat.reshape(B, NK, fw), wf, o)
        m, v = _combine_stats(st[:, 0], st[:, 1], st[:, 2], float(NK))
        yms.append(ymax)
        ms.append(m.reshape(1, -1))
        svs.append(jnp.sqrt(v + EPS).reshape(1, -1))
        gs.append(params[gn].reshape(1, -1))
        bs.append(params[bnm].reshape(1, -1))

    stats5 = _conv5_call(yms, ms, svs, gs, bs, params['W5'].T)
    ymax5 = stats5[:, 0, :]
    m5, v5 = _combine_stats(stats5[:, 1], stats5[:, 2], stats5[:, 3], float(N))
    return _fc_call(ymax5, m5.reshape(1, -1), jnp.sqrt(v5 + EPS).reshape(1, -1),
                    params)


# speedup vs baseline: 12.0359x; 1.0068x over previous
import functools

import jax
import jax.numpy as jnp
from jax import lax
from jax.experimental import pallas as pl
from jax.experimental.pallas import tpu as pltpu
from jax.experimental.pallas import tpu_sc as plsc

KNN = 10
N = 1024
B = 32
P = B * N
NK = N * KNN
DP = 128
EPS = 1e-5
NW = 32
CCH = 16
CK = CCH * KNN


def _lrelu(x):
    return jnp.where(x >= 0, x, 0.2 * x)


def _dot(a, b, dims, prec=None):
    return lax.dot_general(a, b, dimension_numbers=(dims, ((), ())),
                           preferred_element_type=jnp.float32,
                           precision=prec)


def _bn_act(x, m_ref, sv_ref, g_ref, b_ref):
    return _lrelu((x - m_ref[...]) / sv_ref[...] * g_ref[...] + b_ref[...])


def _prep_core(xp, b):
    n = xp.shape[0]
    g_def = _dot(xp, xp, ((1,), (1,)))
    coli = lax.broadcasted_iota(jnp.int32, (n, n), 1)
    sqc = jnp.sum(xp * xp, axis=1, keepdims=True)
    sqr = jnp.transpose(sqc)
    d = (sqc + sqr) - 2.0 * g_def
    idx16 = jnp.zeros((n, 16), jnp.int32)
    lane16 = lax.broadcasted_iota(jnp.int32, (n, 16), 1)
    big = jnp.float32(3.0e38)
    for it in range(KNN + 1):
        rmin = jnp.min(d, axis=1, keepdims=True)
        amin = jnp.min(jnp.where(d == rmin, coli, n), axis=1, keepdims=True)
        if it > 0:
            idx16 = jnp.where(lane16 == (it - 1), amin, idx16)
        d = jnp.where(coli == amin, big, d)
    return idx16 + b * jnp.int32(N)


def _pad_dp(x):
    n, dd = x.shape
    if dd < DP:
        x = jnp.concatenate([x, jnp.zeros((n, DP - dd), jnp.float32)], axis=1)
    return x


def _prep_body_first(x_ref, xout_ref, idx_ref):
    b = pl.program_id(0)
    xp = _pad_dp(x_ref[0])
    xout_ref[0] = xp
    idx_ref[0] = _prep_core(xp, b)


def _prep_body_act(ym_ref, m_ref, sv_ref, g_ref, b_ref, xout_ref, idx_ref):
    b = pl.program_id(0)
    xp = _pad_dp(_bn_act(ym_ref[0], m_ref, sv_ref, g_ref, b_ref))
    xout_ref[0] = xp
    idx_ref[0] = _prep_core(xp, b)


def _prep_call(inputs, d, first):
    body = _prep_body_first if first else _prep_body_act
    in_specs = [pl.BlockSpec((1, N, d), lambda b: (b, 0, 0))]
    if not first:
        in_specs += [pl.BlockSpec((1, d), lambda b: (0, 0))] * 4
    return pl.pallas_call(
        body,
        grid=(B,),
        in_specs=in_specs,
        out_specs=[pl.BlockSpec((1, N, DP), lambda b: (b, 0, 0)),
                   pl.BlockSpec((1, N, 16), lambda b: (b, 0, 0))],
        out_shape=[jax.ShapeDtypeStruct((B, N, DP), jnp.float32),
                   jax.ShapeDtypeStruct((B, N, 16), jnp.int32)],
    )(*inputs)


def _sc_gather_edges(x_flat, idx_flat, fw):
    half = fw // 2
    ppw = P // NW
    nch = ppw // CCH
    subs = []
    off = 0
    while off < CK:
        ln = min(128, CK - off)
        subs.append((off, ln))
        off += ln
    mesh = plsc.VectorSubcoreMesh(core_axis_name="c", subcore_axis_name="s")

    @functools.partial(
        pl.kernel, mesh=mesh,
        out_type=jax.ShapeDtypeStruct((P * KNN, fw), jnp.float32),
        scratch_types=[pltpu.VMEM((2 * CK,), jnp.int32),
                       pltpu.VMEM((2 * CK, DP), jnp.float32),
                       pltpu.VMEM((2 * CCH, DP), jnp.float32),
                       pltpu.VMEM((CK, fw), jnp.float32),
                       pltpu.SemaphoreType.DMA,
                       pltpu.SemaphoreType.DMA],
    )
    def k(x_hbm, idx_hbm, f_hbm, idx_v, rows_v, ctr_v, f_v, sem_a, sem_b):
        wid = lax.axis_index("s") * 2 + lax.axis_index("c")
        sems = (sem_a, sem_b)

        def fire(g, par):
            p0 = wid * ppw + g * CCH
            pltpu.sync_copy(idx_hbm.at[pl.ds(p0 * KNN, CK)],
                            idx_v.at[pl.ds(par * CK, CK)])
            for (soff, sln) in subs:
                pltpu.async_copy(
                    x_hbm.at[idx_v.at[pl.ds(par * CK + soff, sln)]],
                    rows_v.at[pl.ds(par * CK + soff, sln)], sems[par])
            pltpu.sync_copy(x_hbm.at[pl.ds(p0, CCH)],
                            ctr_v.at[pl.ds(par * CCH, CCH)])

        def drain(par):
            for (soff, sln) in subs:
                pltpu.make_async_copy(
                    x_hbm.at[idx_v.at[pl.ds(par * CK + soff, sln)]],
                    rows_v.at[pl.ds(par * CK + soff, sln)],
                    sems[par]).wait()

        def consume(g, par):
            drain(par)

            def pt(i, c2):
                for v in range(half // 16):
                    sl = pl.ds(v * 16, 16)
                    sh = pl.ds(half + v * 16, 16)
                    c16 = ctr_v[par * CCH + i, sl]
                    for j in range(KNN):
                        e = i * KNN + j
                        f_v[e, sl] = rows_v[par * CK + e, sl] - c16
                        f_v[e, sh] = c16
                return c2

            lax.fori_loop(0, CCH, pt, 0)
            p0 = wid * ppw + g * CCH
            pltpu.sync_copy(f_v, f_hbm.at[pl.ds(p0 * KNN, CK)])

        fire(0, 0)

        def pair(gg, carry):
            g0 = 2 * gg
            fire(g0 + 1, 1)
            consume(g0, 0)

            @pl.when(g0 + 2 < nch)
            def _():
                fire(g0 + 2, 0)

            consume(g0 + 1, 1)
            return carry

        lax.fori_loop(0, nch // 2, pair, 0)

    return k(x_flat, idx_flat)


def _edge_body(f_ref, wf_ref, ym_ref, st_ref):
    y = _dot(f_ref[0], wf_ref[...], ((1,), (0,)))
    o = y.shape[1]
    ym_ref[0] = jnp.max(y.reshape(N, KNN, o), axis=1)
    s1 = jnp.sum(y, axis=0, keepdims=True)
    c = s1 / float(NK)
    s2c = jnp.sum((y - c) ** 2, axis=0, keepdims=True)
    st_ref[0] = jnp.concatenate([s1, s2c, c,
                                 jnp.zeros((5, o), jnp.float32)], axis=0)


def _edge_call(f3, wf, o):
    fw = wf.shape[0]
    return pl.pallas_call(
        _edge_body,
        grid=(B,),
        in_specs=[pl.BlockSpec((1, NK, fw), lambda b: (b, 0, 0)),
                  pl.BlockSpec((fw, o), lambda b: (0, 0))],
        out_specs=[pl.BlockSpec((1, N, o), lambda b: (b, 0, 0)),
                   pl.BlockSpec((1, 8, o), lambda b: (b, 0, 0))],
        out_shape=[jax.ShapeDtypeStruct((B, N, o), jnp.float32),
                   jax.ShapeDtypeStruct((B, 8, o), jnp.float32)],
    )(f3, wf)


def _conv5_body(y1, m1, v1, g1, b1, y2, m2, v2, g2, b2,
                y3, m3, v3, g3, b3, y4, m4, v4, g4, b4, w5t_ref, out_ref):
    xs = [_bn_act(y1[0], m1, v1, g1, b1), _bn_act(y2[0], m2, v2, g2, b2),
          _bn_act(y3[0], m3, v3, g3, b3), _bn_act(y4[0], m4, v4, g4, b4)]
    xcat = jnp.concatenate(xs, axis=1)
    y = _dot(xcat, w5t_ref[...], ((1,), (0,)))
    ymax = jnp.max(y, axis=0, keepdims=True)
    s1 = jnp.sum(y, axis=0, keepdims=True)
    c = s1 / float(N)
    s2c = jnp.sum((y - c) ** 2, axis=0, keepdims=True)
    out_ref[0] = jnp.concatenate(
        [ymax, s1, s2c, c, jnp.zeros((4, y.shape[1]), jnp.float32)], axis=0)


def _conv5_call(yms, ms, svs, gs, bs, w5t):
    dims = [64, 64, 128, 256]
    in_specs = []
    inputs = []
    for l in range(4):
        d = dims[l]
        in_specs += [pl.BlockSpec((1, N, d), lambda b: (b, 0, 0))]
        in_specs += [pl.BlockSpec((1, d), lambda b: (0, 0))] * 4
        inputs += [yms[l], ms[l], svs[l], gs[l], bs[l]]
    in_specs += [pl.BlockSpec((512, 1024), lambda b: (0, 0))]
    inputs += [w5t]
    return pl.pallas_call(
        _conv5_body,
        grid=(B,),
        in_specs=in_specs,
        out_specs=pl.BlockSpec((1, 8, 1024), lambda b: (b, 0, 0)),
        out_shape=jax.ShapeDtypeStruct((B, 8, 1024), jnp.float32),
    )(*inputs)


def _fc_body(ym_ref, m5_ref, v5_ref, g5_ref, b5_ref, w1_ref, g1_ref, b1_ref,
             w2_ref, c2b_ref, g2_ref, b2_ref, w3_ref, c3b_ref, out_ref):
    y = _bn_act(ym_ref[...], m5_ref, v5_ref, g5_ref, b5_ref)
    h = _dot(y, w1_ref[...], ((1,), (0,)))
    m = jnp.mean(h, axis=0, keepdims=True)
    v = jnp.mean((h - m) ** 2, axis=0, keepdims=True)
    h = _lrelu((h - m) / jnp.sqrt(v + EPS) * g1_ref[...] + b1_ref[...])
    h = _dot(h, w2_ref[...], ((1,), (0,))) + c2b_ref[...]
    m = jnp.mean(h, axis=0, keepdims=True)
    v = jnp.mean((h - m) ** 2, axis=0, keepdims=True)
    h = _lrelu((h - m) / jnp.sqrt(v + EPS) * g2_ref[...] + b2_ref[...])
    out_ref[...] = _dot(h, w3_ref[...], ((1,), (0,))) + c3b_ref[...]


def _fc_call(ymax, m5, sv5, params):
    w3t = jnp.zeros((256, 128), jnp.float32).at[:, :40].set(params['fW3'].T)
    c3b = jnp.zeros((1, 128), jnp.float32).at[:, :40].set(
        params['fc3b'].reshape(1, -1))
    out = pl.pallas_call(
        _fc_body,
        out_shape=jax.ShapeDtypeStruct((B, 128), jnp.float32),
    )(ymax, m5, sv5, params['g5'].reshape(1, -1), params['b5'].reshape(1, -1),
      params['fW1'].T, params['fg1'].reshape(1, -1), params['fb1'].reshape(1, -1),
      params['fW2'].T, params['fc2b'].reshape(1, -1),
      params['fg2'].reshape(1, -1), params['fb2'].reshape(1, -1),
      w3t, c3b)
    return out[:, :40]


def _combine_stats(s1_b, s2c_b, c_b, nb):
    t = nb * B
    m = jnp.sum(s1_b, axis=0) / t
    r_b = s1_b - nb * c_b
    dc = c_b - m[None, :]
    v = jnp.sum(s2c_b + 2.0 * dc * r_b + nb * dc * dc, axis=0) / t
    return m, v


def _pack_w(w, d, fw):
    half = fw // 2
    o = w.shape[0]
    wf = jnp.zeros((fw, o), jnp.float32)
    wf = wf.at[:d, :].set(w[:, :d].T)
    wf = wf.at[half:half + d, :].set(w[:, d:].T)
    return wf


_LAYERS = (('W1', 'g1', 'b1', 3, 64, 32),
           ('W2', 'g2', 'b2', 64, 64, 128),
           ('W3', 'g3', 'b3', 64, 128, 128),
           ('W4', 'g4', 'b4', 128, 256, 256))


def kernel(points, params):
    yms, ms, svs, gs, bs = [], [], [], [], []
    xout = idx16 = None
    for li, (wn, gn, bnm, d, o, fw) in enumerate(_LAYERS):
        if li == 0:
            xout, idx16 = _prep_call((points,), d, True)
        else:
            xout, idx16 = _prep_call(
                (yms[-1], ms[-1], svs[-1], gs[-1], bs[-1]), d, False)
        idx_flat = idx16[:, :, :KNN].reshape(-1)
        f_flat = _sc_gather_edges(xout.reshape(P, DP), idx_flat, fw)
        wf = _pack_w(params[wn], d, fw)
        ymax, st = _edge_call(f_flat.reshape(B, NK, fw), wf, o)
        m, v = _combine_stats(st[:, 0], st[:, 1], st[:, 2], float(NK))
        yms.append(ymax)
        ms.append(m.reshape(1, -1))
        svs.append(jnp.sqrt(v + EPS).reshape(1, -1))
        gs.append(params[gn].reshape(1, -1))
        bs.append(params[bnm].reshape(1, -1))

    stats5 = _conv5_call(yms, ms, svs, gs, bs, params['W5'].T)
    ymax5 = stats5[:, 0, :]
    m5, v5 = _combine_stats(stats5[:, 1], stats5[:, 2], stats5[:, 3], float(N))
    return _fc_call(ymax5, m5.reshape(1, -1), jnp.sqrt(v5 + EPS).reshape(1, -1),
                    params)
